# Initial kernel scaffold; baseline (speedup 1.0000x reference)
import jax
import jax.numpy as jnp
from jax.experimental import pallas as pl


def kernel(X_t, A_t, bond_src, bond_dst, B_t, Xp, Hp, lig_mask, pocket_mask, edge_mask, t, params):
    raise NotImplementedError("write your pallas kernel here")


# speedup vs baseline: 206.3957x; 206.3957x over previous
import jax
import jax.numpy as jnp
from jax import lax
from jax.experimental import pallas as pl
from jax.experimental.pallas import tpu as pltpu

B, NL, E, NP_, KA, KB, DP, H, L, KC = 32, 64, 128, 512, 16, 5, 128, 128, 4, 16
_F32 = jnp.float32


def _onehot_T(row, n):
    m = row.shape[1]
    ri = jnp.broadcast_to(row, (n, m))
    ki = lax.broadcasted_iota(jnp.int32, (n, m), 0)
    return (ri == ki).astype(_F32)


def _tdot(a, b):
    return lax.dot_general(a, b, (((0,), (0,)), ((), ())),
                           preferred_element_type=_F32)


def _mm(a, b):
    return jnp.dot(a, b, preferred_element_type=_F32)


def _body(xt_ref, xpT_ref, hp_ref, a_ref, bsrc_ref, bdst_ref, bt_ref,
          ligm_ref, pockm_ref, edgem_ref, t_ref,
          atom_emb, bond_emb, pocket_W, pocket_b, coord_W, coord_b,
          t_W, t_b,
          gru_Wih, gru_Whh, gru_bih, gru_bhh,
          msg_W1, msg_b1, msg_W2, msg_b2,
          cross_W1, cross_b1, cross_W2, cross_b2,
          eps_W1, eps_b1, eps_W2, eps_b2,
          A_W1, A_b1, A_W2, A_b2,
          B_W1, B_b1, B_W2, B_b2,
          eps_out, a0_out, b0_out,
          hpk_ref, gc_ref):
    b = pl.program_id(0)
    Xb = xt_ref[0]
    XpT = xpT_ref[0]
    Hpb = hp_ref[0]
    arow = a_ref[0]
    srow = bsrc_ref[0]
    drow = bdst_ref[0]
    brow = bt_ref[0]
    ligm = ligm_ref[0]
    pockm = pockm_ref[0]
    edgem = edgem_ref[0]

    tf = t_ref[...]
    tmax = jnp.maximum(1.0, jnp.max(tf))
    lane = lax.broadcasted_iota(jnp.int32, (1, B), 1)
    tb = jnp.sum(jnp.where(lane == b, tf, 0.0))
    tn = tb / tmax
    half = H // 2
    kf = lax.broadcasted_iota(jnp.int32, (1, half), 1).astype(_F32)
    freqs = jnp.exp(-kf * (jnp.log(10000.0) / (half - 1)))
    args = tn * freqs
    htrow = jnp.concatenate([jnp.sin(args), jnp.cos(args)], axis=1)
    htrow = _mm(htrow, t_W[...]) + t_b[...]

    ohA = _onehot_T(jnp.clip(arow, 0, KA - 1), KA)
    ohBt = _onehot_T(jnp.clip(brow, 0, KB - 1), KB)
    ohS = _onehot_T(srow, NL)
    ohD = _onehot_T(drow, NL)

    hL = (_tdot(ohA, atom_emb[...]) + _mm(Xb, coord_W[...])
          + coord_b[...] + htrow)
    hP = _mm(Hpb, pocket_W[...]) + pocket_b[...]
    hb = _tdot(ohBt, bond_emb[...])

    sx = _tdot(ohS, Xb)
    dx = _tdot(ohD, Xb)
    rel = dx - sx
    dist = jnp.sqrt(jnp.maximum(jnp.sum(rel * rel, axis=1, keepdims=True),
                                1e-12))
    dist = jnp.maximum(dist, 1e-6)
    geom = jnp.concatenate([dist, rel / dist], axis=1)

    d2 = jnp.zeros((NL, NP_), _F32)
    for c in range(3):
        diff = Xb[:, c:c + 1] - XpT[c:c + 1, :]
        d2 = d2 + diff * diff
    dmat = jnp.sqrt(jnp.maximum(d2, 1e-12))
    masked = dmat + (1.0 - pockm) * 1000000.0
    colid = lax.broadcasted_iota(jnp.int32, (NL, NP_), 1)
    for k in range(KC):
        mn = jnp.min(masked, axis=1, keepdims=True)
        idx = jnp.min(jnp.where(masked <= mn, colid, NP_), axis=1,
                      keepdims=True)
        hit = colid == idx
        sel = hit.astype(_F32)
        masked = jnp.where(hit, 1e30, masked)
        hpk_ref[pl.ds(k * NL, NL), :] = _mm(sel, hP)
        xpk = lax.dot_general(sel, XpT, (((1,), (1,)), ((), ())),
                              preferred_element_type=_F32)
        relk = xpk - Xb
        dk = jnp.sqrt(jnp.maximum(jnp.sum(relk * relk, axis=1,
                                          keepdims=True), 1e-12))
        dk = jnp.maximum(dk, 1e-6)
        gc_ref[pl.ds(k * NL, NL), :] = jnp.concatenate([dk, relk / dk],
                                                       axis=1)

    gc = gc_ref[...]
    w = jnp.minimum(1.0 / gc[:, 0:1], 10.0)
    hpk = hpk_ref[...]

    for l in range(L):
        h_src = _tdot(ohS, hL)
        h_dst = _tdot(ohD, hL)
        msg_in = jnp.concatenate([h_src, h_dst, hb, geom], axis=1)
        m1 = jnp.maximum(_mm(msg_in, msg_W1[l]) + msg_b1[l:l + 1, :], 0.0)
        m2 = _mm(m1, msg_W2[l]) + msg_b2[l:l + 1, :]
        msg = m2 * edgem
        agg = _mm(ohD, msg)
        gi = _mm(agg, gru_Wih[l]) + gru_bih[l:l + 1, :]
        gh = _mm(hL, gru_Whh[l]) + gru_bhh[l:l + 1, :]
        r = jax.nn.sigmoid(gi[:, :H] + gh[:, :H])
        z = jax.nn.sigmoid(gi[:, H:2 * H] + gh[:, H:2 * H])
        n = jnp.tanh(gi[:, 2 * H:] + r * gh[:, 2 * H:])
        h_new = (1.0 - z) * n + z * hL

        hq = jnp.concatenate([h_new] * KC, axis=0)
        big = jnp.concatenate([hq, hpk, gc], axis=1)
        c1 = jnp.maximum(_mm(big, cross_W1[l]) + cross_b1[l:l + 1, :], 0.0)
        c2 = _mm(c1, cross_W2[l]) + cross_b2[l:l + 1, :]
        cmw = c2 * w
        cross_agg = cmw[0:NL, :]
        for k in range(1, KC):
            cross_agg = cross_agg + cmw[k * NL:(k + 1) * NL, :]
        hL = (h_new + cross_agg) * ligm

    e1 = jnp.maximum(_mm(hL, eps_W1[...]) + eps_b1[...], 0.0)
    eps_out[0] = _mm(e1, eps_W2[...]) + eps_b2[...]
    a1 = jnp.maximum(_mm(hL, A_W1[...]) + A_b1[...], 0.0)
    a0_out[0] = _mm(a1, A_W2[...]) + A_b2[...]
    h_src = _tdot(ohS, hL)
    h_dst = _tdot(ohD, hL)
    e_in = jnp.concatenate([h_src, h_dst, hb, geom], axis=1)
    b1v = jnp.maximum(_mm(e_in, B_W1[...]) + B_b1[...], 0.0)
    b0_out[0] = _mm(b1v, B_W2[...]) + B_b2[...]


def kernel(X_t, A_t, bond_src, bond_dst, B_t, Xp, Hp, lig_mask,
           pocket_mask, edge_mask, t, params):
    p = params
    XpT = jnp.swapaxes(Xp, 1, 2)
    i32 = jnp.int32
    A3 = A_t.astype(i32).reshape(B, 1, NL)
    S3 = bond_src.astype(i32).reshape(B, 1, E)
    D3 = bond_dst.astype(i32).reshape(B, 1, E)
    Bt3 = B_t.astype(i32).reshape(B, 1, E)
    lig3 = lig_mask.reshape(B, NL, 1)
    pock3 = pocket_mask.reshape(B, 1, NP_)
    edge3 = edge_mask.reshape(B, E, 1)
    t2 = t.astype(_F32).reshape(1, B)

    def row(v):
        return v.reshape(1, -1)

    per_b = lambda *trail: pl.BlockSpec((1,) + trail,
                                        lambda b: (b,) + (0,) * len(trail))
    full = lambda shape: pl.BlockSpec(shape, lambda b: (0,) * len(shape))

    operands = [
        X_t, XpT, Hp, A3, S3, D3, Bt3, lig3, pock3, edge3, t2,
        p['atom_emb'], p['bond_emb'], p['pocket_W'], row(p['pocket_b']),
        p['coord_W'], row(p['coord_b']), p['t_W'], row(p['t_b']),
        p['gru_Wih'], p['gru_Whh'], p['gru_bih'], p['gru_bhh'],
        p['msg_W1'], p['msg_b1'], p['msg_W2'], p['msg_b2'],
        p['cross_W1'], p['cross_b1'], p['cross_W2'], p['cross_b2'],
        p['eps_W1'], row(p['eps_b1']), p['eps_W2'], row(p['eps_b2']),
        p['A_W1'], row(p['A_b1']), p['A_W2'], row(p['A_b2']),
        p['B_W1'], row(p['B_b1']), p['B_W2'], row(p['B_b2']),
    ]
    in_specs = [
        per_b(NL, 3), per_b(3, NP_), per_b(NP_, DP), per_b(1, NL),
        per_b(1, E), per_b(1, E), per_b(1, E), per_b(NL, 1),
        per_b(1, NP_), per_b(E, 1), full((1, B)),
    ] + [full(op.shape) for op in operands[11:]]

    out_shapes = (
        jax.ShapeDtypeStruct((B, NL, 3), _F32),
        jax.ShapeDtypeStruct((B, NL, KA), _F32),
        jax.ShapeDtypeStruct((B, E, KB), _F32),
    )
    out_specs = (per_b(NL, 3), per_b(NL, KA), per_b(E, KB))

    eps, a0, b0 = pl.pallas_call(
        _body,
        grid=(B,),
        in_specs=in_specs,
        out_specs=out_specs,
        out_shape=out_shapes,
        scratch_shapes=[
            pltpu.VMEM((KC * NL, H), _F32),
            pltpu.VMEM((KC * NL, 4), _F32),
        ],
        compiler_params=pltpu.CompilerParams(
            dimension_semantics=("arbitrary",),
        ),
    )(*operands)
    return (eps, a0, b0)


# speedup vs baseline: 330.9312x; 1.6034x over previous
import jax
import jax.numpy as jnp
from jax import lax
from jax.experimental import pallas as pl
from jax.experimental.pallas import tpu as pltpu

B, NL, E, NP_, KA, KB, DP, H, L, KC = 32, 64, 128, 512, 16, 5, 128, 128, 4, 16
NB = 4
NN = NB * NL
NE = NB * E
NX = KC * NN
_F32 = jnp.float32


def _onehot_T(row, n):
    m = row.shape[1]
    ri = jnp.broadcast_to(row, (n, m))
    ki = lax.broadcasted_iota(jnp.int32, (n, m), 0)
    return (ri == ki).astype(_F32)


def _tdot(a, b):
    return lax.dot_general(a, b, (((0,), (0,)), ((), ())),
                           preferred_element_type=_F32)


def _mm(a, b):
    return jnp.dot(a, b, preferred_element_type=_F32)


def _body(xt_ref, xpT_ref, hp_ref, a_ref, bsrc_ref, bdst_ref, bt_ref,
          ligm_ref, pockm_ref, edgem_ref, t_ref,
          atom_emb, bond_emb, pocket_W, pocket_b, coord_W, coord_b,
          t_W, t_b,
          gru_Wih, gru_Whh, gru_bih, gru_bhh,
          msg_W1, msg_b1, msg_W2, msg_b2,
          cross_W1, cross_b1, cross_W2, cross_b2,
          eps_W1, eps_b1, eps_W2, eps_b2,
          A_W1, A_b1, A_W2, A_b2,
          B_W1, B_b1, B_W2, B_b2,
          eps_out, a0_out, b0_out,
          hpk_ref, gc_ref):
    g = pl.program_id(0)
    X_all = xt_ref[0]
    arow = a_ref[0]
    srow = bsrc_ref[0]
    drow = bdst_ref[0]
    brow = bt_ref[0]
    ligm = ligm_ref[0]
    edgem = edgem_ref[0]

    tf = t_ref[...]
    tmax = jnp.maximum(1.0, jnp.max(tf))
    lane = lax.broadcasted_iota(jnp.int32, (1, B), 1)
    half = H // 2
    kf = lax.broadcasted_iota(jnp.int32, (1, half), 1).astype(_F32)
    freqs = jnp.exp(-kf * (jnp.log(10000.0) / (half - 1)))
    ht_parts = []
    for i in range(NB):
        tb = jnp.sum(jnp.where(lane == g * NB + i, tf, 0.0))
        args = (tb / tmax) * freqs
        htrow = jnp.concatenate([jnp.sin(args), jnp.cos(args)], axis=1)
        htrow = _mm(htrow, t_W[...]) + t_b[...]
        ht_parts.append(jnp.broadcast_to(htrow, (NL, H)))
    ht_all = jnp.concatenate(ht_parts, axis=0)

    ohA = _onehot_T(jnp.clip(arow, 0, KA - 1), KA)
    ohBt = _onehot_T(jnp.clip(brow, 0, KB - 1), KB)
    eoff = (lax.broadcasted_iota(jnp.int32, (1, NE), 1) // E) * NL
    ohS = _onehot_T(srow + eoff, NN)
    ohD = _onehot_T(drow + eoff, NN)

    hL = (_tdot(ohA, atom_emb[...]) + _mm(X_all, coord_W[...])
          + coord_b[...] + ht_all)
    hP = _mm(hp_ref[0], pocket_W[...]) + pocket_b[...]
    hb = _tdot(ohBt, bond_emb[...])

    sx = _tdot(ohS, X_all)
    dx = _tdot(ohD, X_all)
    rel = dx - sx
    dist = jnp.sqrt(jnp.maximum(jnp.sum(rel * rel, axis=1, keepdims=True),
                                1e-12))
    dist = jnp.maximum(dist, 1e-6)
    geom = jnp.concatenate([dist, rel / dist], axis=1)

    xp_rows = []
    pm_rows = []
    for i in range(NB):
        xp_rows.append([jnp.broadcast_to(xpT_ref[0, i * 3 + c:i * 3 + c + 1, :],
                                         (NL, NP_)) for c in range(3)])
        pm_rows.append(jnp.broadcast_to(pockm_ref[i], (NL, NP_)))
    XP = [jnp.concatenate([xp_rows[i][c] for i in range(NB)], axis=0)
          for c in range(3)]
    PM = jnp.concatenate(pm_rows, axis=0)
    d2 = jnp.zeros((NN, NP_), _F32)
    for c in range(3):
        diff = X_all[:, c:c + 1] - XP[c]
        d2 = d2 + diff * diff
    masked = jnp.sqrt(jnp.maximum(d2, 1e-12)) + (1.0 - PM) * 1000000.0

    colid = lax.broadcasted_iota(jnp.int32, (NN, NP_), 1)
    idxs = []
    for k in range(KC):
        mn = jnp.min(masked, axis=1, keepdims=True)
        idx = jnp.min(jnp.where(masked <= mn, colid, NP_), axis=1,
                      keepdims=True)
        idxs.append(idx)
        masked = jnp.where(colid == idx, 1e30, masked)

    cross_id = lax.broadcasted_iota(jnp.int32, (KC * NL, NP_), 1)
    for i in range(NB):
        idxcat = jnp.concatenate(
            [idxs[k][i * NL:(i + 1) * NL] for k in range(KC)], axis=0)
        sel = (cross_id == idxcat).astype(_F32)
        hpk_i = _mm(sel, hP[i * NP_:(i + 1) * NP_])
        xpk_i = lax.dot_general(
            sel, xpT_ref[0, i * 3:(i + 1) * 3], (((1,), (1,)), ((), ())),
            preferred_element_type=_F32)
        xtile = jnp.concatenate([X_all[i * NL:(i + 1) * NL]] * KC, axis=0)
        relk = xpk_i - xtile
        dk = jnp.sqrt(jnp.maximum(jnp.sum(relk * relk, axis=1,
                                          keepdims=True), 1e-12))
        dk = jnp.maximum(dk, 1e-6)
        gck = jnp.concatenate([dk, relk / dk], axis=1)
        for k in range(KC):
            r0 = k * NN + i * NL
            hpk_ref[pl.ds(r0, NL), :] = hpk_i[k * NL:(k + 1) * NL]
            gc_ref[pl.ds(r0, NL), :] = gck[k * NL:(k + 1) * NL]

    gc = gc_ref[...]
    w = jnp.minimum(1.0 / gc[:, 0:1], 10.0)
    hpk = hpk_ref[...]

    for l in range(L):
        h_src = _tdot(ohS, hL)
        h_dst = _tdot(ohD, hL)
        msg_in = jnp.concatenate([h_src, h_dst, hb, geom], axis=1)
        m1 = jnp.maximum(_mm(msg_in, msg_W1[l]) + msg_b1[l:l + 1, :], 0.0)
        m2 = _mm(m1, msg_W2[l]) + msg_b2[l:l + 1, :]
        msg = m2 * edgem
        agg = _mm(ohD, msg)
        gi = _mm(agg, gru_Wih[l]) + gru_bih[l:l + 1, :]
        gh = _mm(hL, gru_Whh[l]) + gru_bhh[l:l + 1, :]
        r = jax.nn.sigmoid(gi[:, :H] + gh[:, :H])
        z = jax.nn.sigmoid(gi[:, H:2 * H] + gh[:, H:2 * H])
        n = jnp.tanh(gi[:, 2 * H:] + r * gh[:, 2 * H:])
        h_new = (1.0 - z) * n + z * hL

        hq = jnp.concatenate([h_new] * KC, axis=0)
        big = jnp.concatenate([hq, hpk, gc], axis=1)
        c1 = jnp.maximum(_mm(big, cross_W1[l]) + cross_b1[l:l + 1, :], 0.0)
        c2 = _mm(c1, cross_W2[l]) + cross_b2[l:l + 1, :]
        cmw = c2 * w
        cross_agg = cmw[0:NN, :]
        for k in range(1, KC):
            cross_agg = cross_agg + cmw[k * NN:(k + 1) * NN, :]
        hL = (h_new + cross_agg) * ligm

    e1 = jnp.maximum(_mm(hL, eps_W1[...]) + eps_b1[...], 0.0)
    eps_out[0] = _mm(e1, eps_W2[...]) + eps_b2[...]
    a1 = jnp.maximum(_mm(hL, A_W1[...]) + A_b1[...], 0.0)
    a0_out[0] = _mm(a1, A_W2[...]) + A_b2[...]
    h_src = _tdot(ohS, hL)
    h_dst = _tdot(ohD, hL)
    e_in = jnp.concatenate([h_src, h_dst, hb, geom], axis=1)
    b1v = jnp.maximum(_mm(e_in, B_W1[...]) + B_b1[...], 0.0)
    b0_out[0] = _mm(b1v, B_W2[...]) + B_b2[...]


def kernel(X_t, A_t, bond_src, bond_dst, B_t, Xp, Hp, lig_mask,
           pocket_mask, edge_mask, t, params):
    p = params
    NG = B // NB
    XpT = jnp.swapaxes(Xp, 1, 2).reshape(NG, NB * 3, NP_)
    i32 = jnp.int32
    X3 = X_t.reshape(NG, NN, 3)
    Hp3 = Hp.reshape(NG, NB * NP_, DP)
    A3 = A_t.astype(i32).reshape(NG, 1, NN)
    S3 = bond_src.astype(i32).reshape(NG, 1, NE)
    D3 = bond_dst.astype(i32).reshape(NG, 1, NE)
    Bt3 = B_t.astype(i32).reshape(NG, 1, NE)
    lig3 = lig_mask.reshape(NG, NN, 1)
    pock3 = pocket_mask.reshape(B, 1, NP_)
    edge3 = edge_mask.reshape(NG, NE, 1)
    t2 = t.astype(_F32).reshape(1, B)

    def row(v):
        return v.reshape(1, -1)

    per_g = lambda *trail: pl.BlockSpec((NB,) + trail,
                                        lambda g: (g,) + (0,) * len(trail))
    full = lambda shape: pl.BlockSpec(shape, lambda g: (0,) * len(shape))

    operands = [
        X3, XpT, Hp3, A3, S3, D3, Bt3, lig3, pock3, edge3, t2,
        p['atom_emb'], p['bond_emb'], p['pocket_W'], row(p['pocket_b']),
        p['coord_W'], row(p['coord_b']), p['t_W'], row(p['t_b']),
        p['gru_Wih'], p['gru_Whh'], p['gru_bih'], p['gru_bhh'],
        p['msg_W1'], p['msg_b1'], p['msg_W2'], p['msg_b2'],
        p['cross_W1'], p['cross_b1'], p['cross_W2'], p['cross_b2'],
        p['eps_W1'], row(p['eps_b1']), p['eps_W2'], row(p['eps_b2']),
        p['A_W1'], row(p['A_b1']), p['A_W2'], row(p['A_b2']),
        p['B_W1'], row(p['B_b1']), p['B_W2'], row(p['B_b2']),
    ]
    one_g = lambda *trail: pl.BlockSpec((1,) + trail,
                                        lambda g: (g,) + (0,) * len(trail))
    in_specs = [
        one_g(NN, 3), one_g(NB * 3, NP_), one_g(NB * NP_, DP), one_g(1, NN),
        one_g(1, NE), one_g(1, NE), one_g(1, NE), one_g(NN, 1),
        per_g(1, NP_), one_g(NE, 1), full((1, B)),
    ] + [full(op.shape) for op in operands[11:]]

    out_shapes = (
        jax.ShapeDtypeStruct((NG, NN, 3), _F32),
        jax.ShapeDtypeStruct((NG, NN, KA), _F32),
        jax.ShapeDtypeStruct((NG, NE, KB), _F32),
    )
    out_specs = (one_g(NN, 3), one_g(NN, KA), one_g(NE, KB))

    eps, a0, b0 = pl.pallas_call(
        _body,
        grid=(B // NB,),
        in_specs=in_specs,
        out_specs=out_specs,
        out_shape=out_shapes,
        scratch_shapes=[
            pltpu.VMEM((NX, H), _F32),
            pltpu.VMEM((NX, 4), _F32),
        ],
        compiler_params=pltpu.CompilerParams(
            dimension_semantics=("arbitrary",),
        ),
    )(*operands)
    return (eps.reshape(B, NL, 3), a0.reshape(B, NL, KA),
            b0.reshape(B, E, KB))
